# baseline (speedup 1.0000x reference)
import functools

import jax
import jax.numpy as jnp
from jax.experimental import pallas as pl
from jax.experimental.pallas import tpu as pltpu

_LOG2E = 1.4426950408889634


def _gat_kernel(nb, x_ref, w_ref, a1_ref, a2_ref, a_ref, acc_ref,
                h_s, f1_s, f2t_s, m_s):
    i = pl.program_id(1)

    @pl.when(i == 0)
    def _proj():
        x = x_ref[0]
        h = jnp.dot(x, w_ref[...], preferred_element_type=jnp.float32,
                    precision=jax.lax.Precision.HIGHEST)
        h_s[...] = h
        f1 = jnp.dot(h, a1_ref[...], preferred_element_type=jnp.float32,
                     precision=jax.lax.Precision.HIGHEST) * _LOG2E
        f2 = jnp.dot(h, a2_ref[...], preferred_element_type=jnp.float32,
                     precision=jax.lax.Precision.HIGHEST) * _LOG2E
        f1_s[...] = f1
        f2t_s[...] = jnp.reshape(f2, f2t_s.shape)
        m_s[...] = jnp.max(f2, axis=0, keepdims=True)
        acc_ref[...] = jnp.sum(x, axis=0, keepdims=True)[None]

    @pl.when(i > 0)
    def _attend():
        blk = a_ref.shape[1]
        r0 = (i - 1) * blk
        f1 = f1_s[pl.ds(r0, blk), :]
        tmax = f1 + m_s[...]
        mrow = jnp.maximum(tmax, 0.2 * tmax)
        t = f1 + f2t_s[...]
        e = jnp.maximum(t, 0.2 * t)
        q = jnp.exp2(e - mrow)
        p = jnp.where(a_ref[0] > 0.0, q, 0.0)
        s = jnp.sum(p, axis=1, keepdims=True)
        s = jnp.maximum(s, jnp.float32(1e-30))
        hp = jnp.dot(p, h_s[...], preferred_element_type=jnp.float32) / s
        contrib = hp + 2.0 * jnp.maximum(hp, 0.0)
        part = jnp.dot(jnp.ones((1, blk), jnp.float32), contrib,
                       preferred_element_type=jnp.float32)
        acc_ref[...] += part[None]


def _mlp_kernel(inv_pool, acc_ref, w1_ref, b1_ref, w2_ref, b2_ref, out_ref):
    xm = acc_ref[...] * inv_pool
    hmid = jnp.dot(xm, w1_ref[...], preferred_element_type=jnp.float32,
                   precision=jax.lax.Precision.HIGHEST) + b1_ref[...]
    hmid = jnp.maximum(hmid, 0.0)
    out_ref[...] = jnp.dot(hmid, w2_ref[...], preferred_element_type=jnp.float32,
                           precision=jax.lax.Precision.HIGHEST) + b2_ref[...]


def kernel(X, A, W_gat, a_gat, W1, b1, W2, b2):
    B, N, F = X.shape
    H = W1.shape[1]
    BLK = 512
    NB = N // BLK
    a1 = a_gat[:F]
    a2 = a_gat[F:]

    acc = pl.pallas_call(
        functools.partial(_gat_kernel, NB),
        grid=(B, NB + 1),
        in_specs=[
            pl.BlockSpec((1, N, F), lambda b, i: (b, 0, 0)),
            pl.BlockSpec((F, F), lambda b, i: (0, 0)),
            pl.BlockSpec((F, 1), lambda b, i: (0, 0)),
            pl.BlockSpec((F, 1), lambda b, i: (0, 0)),
            pl.BlockSpec((1, BLK, N), lambda b, i: (b, jnp.maximum(i, 1) - 1, 0)),
        ],
        out_specs=pl.BlockSpec((1, 1, F), lambda b, i: (b, 0, 0)),
        out_shape=jax.ShapeDtypeStruct((B, 1, F), jnp.float32),
        scratch_shapes=[
            pltpu.VMEM((N, F), jnp.float32),
            pltpu.VMEM((N, 1), jnp.float32),
            pltpu.VMEM((1, N), jnp.float32),
            pltpu.VMEM((1, 1), jnp.float32),
        ],
        compiler_params=pltpu.CompilerParams(
            dimension_semantics=("parallel", "arbitrary"),
        ),
    )(X, W_gat, a1, a2, A)

    out = pl.pallas_call(
        functools.partial(_mlp_kernel, 1.0 / (4.0 * N)),
        in_specs=[
            pl.BlockSpec((B, F), lambda: (0, 0)),
            pl.BlockSpec((F, H), lambda: (0, 0)),
            pl.BlockSpec((1, H), lambda: (0, 0)),
            pl.BlockSpec((H, 1), lambda: (0, 0)),
            pl.BlockSpec((1, 1), lambda: (0, 0)),
        ],
        out_specs=pl.BlockSpec((B, 1), lambda: (0, 0)),
        out_shape=jax.ShapeDtypeStruct((B, 1), jnp.float32),
    )(acc.reshape(B, F), W1, b1.reshape(1, H), W2, b2.reshape(1, 1))

    return out


# speedup vs baseline: 1.1035x; 1.1035x over previous
import functools

import jax
import jax.numpy as jnp
from jax.experimental import pallas as pl
from jax.experimental.pallas import tpu as pltpu

_LOG2E = 1.4426950408889634


def _gat_kernel(nb, x_ref, w_ref, a1_ref, a2_ref, a_ref, acc_ref,
                h_s, f1_s, f2t_s, m_s):
    i = pl.program_id(1)

    @pl.when(i == 0)
    def _proj():
        x = x_ref[0]
        h = jnp.dot(x, w_ref[...], preferred_element_type=jnp.float32,
                    precision=jax.lax.Precision.HIGHEST)
        h_s[...] = h
        f1 = jnp.dot(h, a1_ref[...], preferred_element_type=jnp.float32,
                     precision=jax.lax.Precision.HIGHEST) * _LOG2E
        f2 = jnp.dot(h, a2_ref[...], preferred_element_type=jnp.float32,
                     precision=jax.lax.Precision.HIGHEST) * _LOG2E
        f1_s[...] = f1
        f2t_s[...] = jnp.reshape(f2, f2t_s.shape)
        m_s[...] = jnp.max(f2, axis=0, keepdims=True)
        acc_ref[...] = jnp.sum(x, axis=0, keepdims=True)[None]

    @pl.when(i > 0)
    def _attend():
        blk = a_ref.shape[1]
        r0 = (i - 1) * blk
        f1 = f1_s[pl.ds(r0, blk), :]
        tmax = f1 + m_s[...]
        mrow = jnp.maximum(tmax, 0.2 * tmax)
        t = f1 + f2t_s[...]
        e = jnp.maximum(t, 0.2 * t)
        q = jnp.exp2(e - mrow)
        p = jnp.where(a_ref[0] > 0.0, q, 0.0)
        s = jnp.dot(p.astype(jnp.bfloat16), jnp.ones((t.shape[1], 1), jnp.bfloat16),
                    preferred_element_type=jnp.float32)
        s = jnp.maximum(s, jnp.float32(1e-30))
        hp = jnp.dot(p, h_s[...], preferred_element_type=jnp.float32) / s
        contrib = hp + 2.0 * jnp.maximum(hp, 0.0)
        part = jnp.dot(jnp.ones((1, blk), jnp.float32), contrib,
                       preferred_element_type=jnp.float32)
        acc_ref[...] += part[None]


def _mlp_kernel(inv_pool, acc_ref, w1_ref, b1_ref, w2_ref, b2_ref, out_ref):
    xm = acc_ref[...] * inv_pool
    hmid = jnp.dot(xm, w1_ref[...], preferred_element_type=jnp.float32,
                   precision=jax.lax.Precision.HIGHEST) + b1_ref[...]
    hmid = jnp.maximum(hmid, 0.0)
    out_ref[...] = jnp.dot(hmid, w2_ref[...], preferred_element_type=jnp.float32,
                           precision=jax.lax.Precision.HIGHEST) + b2_ref[...]


def kernel(X, A, W_gat, a_gat, W1, b1, W2, b2):
    B, N, F = X.shape
    H = W1.shape[1]
    BLK = 512
    NB = N // BLK
    a1 = a_gat[:F]
    a2 = a_gat[F:]

    acc = pl.pallas_call(
        functools.partial(_gat_kernel, NB),
        grid=(B, NB + 1),
        in_specs=[
            pl.BlockSpec((1, N, F), lambda b, i: (b, 0, 0)),
            pl.BlockSpec((F, F), lambda b, i: (0, 0)),
            pl.BlockSpec((F, 1), lambda b, i: (0, 0)),
            pl.BlockSpec((F, 1), lambda b, i: (0, 0)),
            pl.BlockSpec((1, BLK, N), lambda b, i: (b, jnp.maximum(i, 1) - 1, 0)),
        ],
        out_specs=pl.BlockSpec((1, 1, F), lambda b, i: (b, 0, 0)),
        out_shape=jax.ShapeDtypeStruct((B, 1, F), jnp.float32),
        scratch_shapes=[
            pltpu.VMEM((N, F), jnp.float32),
            pltpu.VMEM((N, 1), jnp.float32),
            pltpu.VMEM((1, N), jnp.float32),
            pltpu.VMEM((1, 1), jnp.float32),
        ],
        compiler_params=pltpu.CompilerParams(
            dimension_semantics=("parallel", "arbitrary"),
        ),
    )(X, W_gat, a1, a2, A)

    out = pl.pallas_call(
        functools.partial(_mlp_kernel, 1.0 / (4.0 * N)),
        in_specs=[
            pl.BlockSpec((B, F), lambda: (0, 0)),
            pl.BlockSpec((F, H), lambda: (0, 0)),
            pl.BlockSpec((1, H), lambda: (0, 0)),
            pl.BlockSpec((H, 1), lambda: (0, 0)),
            pl.BlockSpec((1, 1), lambda: (0, 0)),
        ],
        out_specs=pl.BlockSpec((B, 1), lambda: (0, 0)),
        out_shape=jax.ShapeDtypeStruct((B, 1), jnp.float32),
    )(acc.reshape(B, F), W1, b1.reshape(1, H), W2, b2.reshape(1, 1))

    return out


# speedup vs baseline: 1.1414x; 1.0344x over previous
import functools

import jax
import jax.numpy as jnp
from jax.experimental import pallas as pl
from jax.experimental.pallas import tpu as pltpu

_LOG2E = 1.4426950408889634


def _gat_kernel(nb, x_ref, w_ref, a1_ref, a2_ref, a_ref, acc_ref,
                h_s, f1_s, f2t_s, m_s):
    i = pl.program_id(1)

    @pl.when(i == 0)
    def _proj():
        x = x_ref[0]
        h = jnp.dot(x, w_ref[...], preferred_element_type=jnp.float32,
                    precision=jax.lax.Precision.HIGHEST)
        h_s[...] = h
        f1 = jnp.dot(h, a1_ref[...], preferred_element_type=jnp.float32,
                     precision=jax.lax.Precision.HIGHEST) * _LOG2E
        f2 = jnp.dot(h, a2_ref[...], preferred_element_type=jnp.float32,
                     precision=jax.lax.Precision.HIGHEST) * _LOG2E
        f1_s[...] = f1
        f2t_s[...] = jnp.reshape(f2, f2t_s.shape)
        m_s[...] = jnp.max(f2, axis=0, keepdims=True)
        acc_ref[...] = jnp.sum(x, axis=0, keepdims=True)[None]

    @pl.when(i > 0)
    def _attend():
        blk = a_ref.shape[1]
        r0 = (i - 1) * blk
        f1 = f1_s[pl.ds(r0, blk), :]
        tmax = f1 + m_s[...]
        mrow = jnp.maximum(tmax, 0.2 * tmax)
        t = f1 + f2t_s[...]
        e = jnp.maximum(t, 0.2 * t)
        q = jnp.exp2(e - mrow)
        p = jnp.where(a_ref[0] > 0.0, q, 0.0)
        s = jnp.dot(p.astype(jnp.bfloat16), jnp.ones((t.shape[1], 1), jnp.bfloat16),
                    preferred_element_type=jnp.float32)
        s = jnp.maximum(s, jnp.float32(1e-30))
        hp = jnp.dot(p, h_s[...], preferred_element_type=jnp.float32) / s
        contrib = hp + 2.0 * jnp.maximum(hp, 0.0)
        part = jnp.dot(jnp.ones((1, blk), jnp.float32), contrib,
                       preferred_element_type=jnp.float32)
        acc_ref[...] += part[None]


def _mlp_kernel(inv_pool, acc_ref, w1_ref, b1_ref, w2_ref, b2_ref, out_ref):
    xm = acc_ref[...] * inv_pool
    hmid = jnp.dot(xm, w1_ref[...], preferred_element_type=jnp.float32,
                   precision=jax.lax.Precision.HIGHEST) + b1_ref[...]
    hmid = jnp.maximum(hmid, 0.0)
    out_ref[...] = jnp.dot(hmid, w2_ref[...], preferred_element_type=jnp.float32,
                           precision=jax.lax.Precision.HIGHEST) + b2_ref[...]


def kernel(X, A, W_gat, a_gat, W1, b1, W2, b2):
    B, N, F = X.shape
    H = W1.shape[1]
    BLK = 1024
    NB = N // BLK
    a1 = a_gat[:F]
    a2 = a_gat[F:]

    acc = pl.pallas_call(
        functools.partial(_gat_kernel, NB),
        grid=(B, NB + 1),
        in_specs=[
            pl.BlockSpec((1, N, F), lambda b, i: (b, 0, 0)),
            pl.BlockSpec((F, F), lambda b, i: (0, 0)),
            pl.BlockSpec((F, 1), lambda b, i: (0, 0)),
            pl.BlockSpec((F, 1), lambda b, i: (0, 0)),
            pl.BlockSpec((1, BLK, N), lambda b, i: (b, jnp.maximum(i, 1) - 1, 0)),
        ],
        out_specs=pl.BlockSpec((1, 1, F), lambda b, i: (b, 0, 0)),
        out_shape=jax.ShapeDtypeStruct((B, 1, F), jnp.float32),
        scratch_shapes=[
            pltpu.VMEM((N, F), jnp.float32),
            pltpu.VMEM((N, 1), jnp.float32),
            pltpu.VMEM((1, N), jnp.float32),
            pltpu.VMEM((1, 1), jnp.float32),
        ],
        compiler_params=pltpu.CompilerParams(
            dimension_semantics=("parallel", "arbitrary"),
            vmem_limit_bytes=100 * 1024 * 1024,
        ),
    )(X, W_gat, a1, a2, A)

    out = pl.pallas_call(
        functools.partial(_mlp_kernel, 1.0 / (4.0 * N)),
        in_specs=[
            pl.BlockSpec((B, F), lambda: (0, 0)),
            pl.BlockSpec((F, H), lambda: (0, 0)),
            pl.BlockSpec((1, H), lambda: (0, 0)),
            pl.BlockSpec((H, 1), lambda: (0, 0)),
            pl.BlockSpec((1, 1), lambda: (0, 0)),
        ],
        out_specs=pl.BlockSpec((B, 1), lambda: (0, 0)),
        out_shape=jax.ShapeDtypeStruct((B, 1), jnp.float32),
    )(acc.reshape(B, F), W1, b1.reshape(1, H), W2, b2.reshape(1, 1))

    return out
